# Initial kernel scaffold; baseline (speedup 1.0000x reference)
import jax
import jax.numpy as jnp
from jax.experimental import pallas as pl


def kernel(x, h, c, g, Wh_a, Wn_a, U_a, u_w, u_b, Va_w, Va_b, Wh, Wn, U, V_w, V_b, neighbor_index, neighbor_mask):
    raise NotImplementedError("write your pallas kernel here")


# speedup vs baseline: 4.9828x; 4.9828x over previous
import functools

import jax
import jax.numpy as jnp
from jax import lax
from jax.experimental import pallas as pl
from jax.experimental.pallas import tpu as pltpu
from jax.experimental.pallas import tpu_sc as plsc

_L = 16


def _make_sc_hbar(B, S, N, H, SB):
    NBLK = S // SB
    PC = H // 32

    @functools.partial(
        pl.kernel,
        out_type=jax.ShapeDtypeStruct((B, NBLK, H, SB), jnp.float32),
        mesh=plsc.VectorSubcoreMesh(core_axis_name="c", subcore_axis_name="s"),
        compiler_params=pltpu.CompilerParams(needs_layout_passes=False),
        scratch_types=[
            pltpu.VMEM(((S + 1) * H,), jnp.float32),
            pltpu.VMEM((SB * N,), jnp.int32),
            pltpu.VMEM((SB * N,), jnp.int32),
            pltpu.VMEM((H, SB), jnp.float32),
        ],
    )
    def sc_hbar(h2, idx2, mask2, out, table, idxv, maskv, outT):
        b = lax.axis_index("s") * 2 + lax.axis_index("c")
        zero16 = jnp.zeros((_L,), jnp.float32)
        for k in range(H // _L):
            table[pl.ds(k * _L, _L)] = zero16
        pltpu.sync_copy(h2.at[b], table.at[pl.ds(H, S * H)])
        iota = lax.iota(jnp.int32, _L)

        def blk_body(blk, carry):
            pltpu.sync_copy(idx2.at[b, pl.ds(blk * SB * N, SB * N)], idxv)
            pltpu.sync_copy(mask2.at[b, pl.ds(blk * SB * N, SB * N)], maskv)

            def mask_body(i, c):
                idxv[pl.ds(i * _L, _L)] = (
                    idxv[pl.ds(i * _L, _L)] * maskv[pl.ds(i * _L, _L)]
                )
                return c

            lax.fori_loop(0, SB * N // _L, mask_body, 0)

            def s16_body(s16, c):
                srow = s16 * _L
                for pc in range(PC):

                    def n_body(n, acc):
                        effcol = plsc.load_gather(idxv, [(srow + iota) * N + n])
                        base = effcol * H
                        return tuple(
                            acc[p] + plsc.load_gather(table, [base + (pc * 32 + p)])
                            for p in range(32)
                        )

                    acc = lax.fori_loop(
                        0, N, n_body, tuple(zero16 for _ in range(32))
                    )
                    for p in range(32):
                        outT[pc * 32 + p, pl.ds(srow, _L)] = acc[p] * (1.0 / N)
                return c

            lax.fori_loop(0, SB // _L, s16_body, 0)
            pltpu.sync_copy(outT, out.at[b, blk])
            return carry

        lax.fori_loop(0, NBLK, blk_body, 0)

    return sc_hbar


def _tc_body(x_ref, h_ref, c_ref, g_ref, hb_ref, Wn_a_ref, Wh_ref, Wn_ref,
             U_ref, V_w_ref, V_b_ref, nh_ref, nc_ref):
    H = h_ref.shape[-1]
    f32 = jnp.float32
    hbt = hb_ref[0, 0]
    hn = lax.dot_general(hbt, Wn_a_ref[...], (((0,), (0,)), ((), ())),
                         preferred_element_type=f32)
    gates = (
        jnp.dot(h_ref[0], Wh_ref[...], preferred_element_type=f32)
        + jnp.dot(x_ref[0], U_ref[...], preferred_element_type=f32)
        + jnp.dot(hn, Wn_ref[...], preferred_element_type=f32)
        + jnp.dot(g_ref[0], V_w_ref[...], preferred_element_type=f32)
        + V_b_ref[...][None, :]
    )
    i_g = gates[:, 0:H]
    f_g = gates[:, H:2 * H]
    o_g = gates[:, 2 * H:3 * H]
    u_g = gates[:, 3 * H:4 * H]
    nc = jax.nn.sigmoid(f_g) * c_ref[0] + jax.nn.sigmoid(i_g) * jnp.tanh(u_g)
    nh_ref[0] = jax.nn.sigmoid(o_g) * jnp.tanh(nc)
    nc_ref[0] = nc


def kernel(x, h, c, g, Wh_a, Wn_a, U_a, u_w, u_b, Va_w, Va_b, Wh, Wn, U,
           V_w, V_b, neighbor_index, neighbor_mask):
    B, S, N = neighbor_index.shape
    H = h.shape[-1]
    IN = x.shape[-1]
    SB = 128
    NBLK = S // SB

    h2 = h.reshape(B, S * H)
    idx2 = neighbor_index.reshape(B, S * N)
    mask2 = neighbor_mask.reshape(B, S * N)
    hbar4 = _make_sc_hbar(B, S, N, H, SB)(h2, idx2, mask2)

    grid = (B, NBLK)
    out_sd = jax.ShapeDtypeStruct((B, S, H), jnp.float32)
    bs3 = lambda shp: pl.BlockSpec(shp, lambda b, sb: (b, sb, 0))
    wfull = lambda shp: pl.BlockSpec(shp, lambda b, sb: tuple(0 for _ in shp))
    new_h, new_c = pl.pallas_call(
        _tc_body,
        grid=grid,
        in_specs=[
            bs3((1, SB, IN)),
            bs3((1, SB, H)),
            bs3((1, SB, H)),
            pl.BlockSpec((1, 1, H), lambda b, sb: (b, 0, 0)),
            pl.BlockSpec((1, 1, H, SB), lambda b, sb: (b, sb, 0, 0)),
            wfull((H, H)),
            wfull((H, 4 * H)),
            wfull((H, 4 * H)),
            wfull((IN, 4 * H)),
            wfull((H, 4 * H)),
            wfull((4 * H,)),
        ],
        out_specs=[bs3((1, SB, H)), bs3((1, SB, H))],
        out_shape=[out_sd, out_sd],
    )(x, h, c, g.reshape(B, 1, H), hbar4, Wn_a, Wh, Wn, U, V_w, V_b)
    return new_h, new_c


# speedup vs baseline: 9.6525x; 1.9372x over previous
import functools

import jax
import jax.numpy as jnp
from jax import lax
from jax.experimental import pallas as pl
from jax.experimental.pallas import tpu as pltpu
from jax.experimental.pallas import tpu_sc as plsc

_L = 16


def _make_sc_hbar(B, S, N, H, SB):
    NBLK = S // SB
    HC = H // _L

    @functools.partial(
        pl.kernel,
        out_type=jax.ShapeDtypeStruct((B, S * H), jnp.float32),
        mesh=plsc.VectorSubcoreMesh(core_axis_name="c", subcore_axis_name="s"),
        compiler_params=pltpu.CompilerParams(needs_layout_passes=False),
        scratch_types=[
            pltpu.VMEM(((S + 1) * H,), jnp.float32),
            pltpu.VMEM((SB * N,), jnp.int32),
            pltpu.VMEM((SB * N,), jnp.int32),
            pltpu.VMEM((SB * H,), jnp.float32),
        ],
    )
    def sc_hbar(h2, idx2, mask2, out, table, idxv, maskv, outv):
        b = lax.axis_index("s") * 2 + lax.axis_index("c")
        zero16 = jnp.zeros((_L,), jnp.float32)
        for k in range(HC):
            table[pl.ds(k * _L, _L)] = zero16
        pltpu.sync_copy(h2.at[b], table.at[pl.ds(H, S * H)])
        iota = lax.iota(jnp.int32, _L)
        ramps = [iota + j * _L for j in range(HC)]

        def blk_body(blk, carry):
            pltpu.sync_copy(idx2.at[b, pl.ds(blk * SB * N, SB * N)], idxv)
            pltpu.sync_copy(mask2.at[b, pl.ds(blk * SB * N, SB * N)], maskv)

            def mask_body(i, c):
                idxv[pl.ds(i * _L, _L)] = (
                    idxv[pl.ds(i * _L, _L)] * maskv[pl.ds(i * _L, _L)]
                )
                return c

            lax.fori_loop(0, SB * N // _L, mask_body, 0)

            def s_body(s, c):
                acc = [zero16] * HC
                for half in range(N // _L):
                    e = idxv[pl.ds(s * N + half * _L, _L)]
                    for n in range(_L):
                        row = jnp.take_along_axis(
                            e, jnp.full((_L,), n, jnp.int32), axis=0,
                            mode="promise_in_bounds")
                        base = row * H
                        for j in range(HC):
                            acc[j] = acc[j] + plsc.load_gather(
                                table, [base + ramps[j]])
                for j in range(HC):
                    outv[pl.ds(s * H + j * _L, _L)] = acc[j] * (1.0 / N)
                return c

            lax.fori_loop(0, SB, s_body, 0)
            pltpu.sync_copy(outv, out.at[b, pl.ds(blk * SB * H, SB * H)])
            return carry

        lax.fori_loop(0, NBLK, blk_body, 0)

    return sc_hbar


def _tc_body(x_ref, h_ref, c_ref, g_ref, hb_ref, Wn_a_ref, Wh_ref, Wn_ref,
             U_ref, V_w_ref, V_b_ref, nh_ref, nc_ref):
    H = h_ref.shape[-1]
    f32 = jnp.float32
    hn = jnp.dot(hb_ref[0], Wn_a_ref[...], preferred_element_type=f32)
    gates = (
        jnp.dot(h_ref[0], Wh_ref[...], preferred_element_type=f32)
        + jnp.dot(x_ref[0], U_ref[...], preferred_element_type=f32)
        + jnp.dot(hn, Wn_ref[...], preferred_element_type=f32)
        + jnp.dot(g_ref[0], V_w_ref[...], preferred_element_type=f32)
        + V_b_ref[...][None, :]
    )
    i_g = gates[:, 0:H]
    f_g = gates[:, H:2 * H]
    o_g = gates[:, 2 * H:3 * H]
    u_g = gates[:, 3 * H:4 * H]
    nc = jax.nn.sigmoid(f_g) * c_ref[0] + jax.nn.sigmoid(i_g) * jnp.tanh(u_g)
    nh_ref[0] = jax.nn.sigmoid(o_g) * jnp.tanh(nc)
    nc_ref[0] = nc


def kernel(x, h, c, g, Wh_a, Wn_a, U_a, u_w, u_b, Va_w, Va_b, Wh, Wn, U,
           V_w, V_b, neighbor_index, neighbor_mask):
    B, S, N = neighbor_index.shape
    H = h.shape[-1]
    IN = x.shape[-1]
    SB = 128
    NBLK = S // SB

    h2 = h.reshape(B, S * H)
    idx2 = neighbor_index.reshape(B, S * N)
    mask2 = neighbor_mask.reshape(B, S * N)
    hbar = _make_sc_hbar(B, S, N, H, SB)(h2, idx2, mask2)
    hbar = hbar.reshape(B, S, H)

    grid = (B, NBLK)
    out_sd = jax.ShapeDtypeStruct((B, S, H), jnp.float32)
    bs3 = lambda shp: pl.BlockSpec(shp, lambda b, sb: (b, sb, 0))
    wfull = lambda shp: pl.BlockSpec(shp, lambda b, sb: tuple(0 for _ in shp))
    new_h, new_c = pl.pallas_call(
        _tc_body,
        grid=grid,
        in_specs=[
            bs3((1, SB, IN)),
            bs3((1, SB, H)),
            bs3((1, SB, H)),
            pl.BlockSpec((1, 1, H), lambda b, sb: (b, 0, 0)),
            bs3((1, SB, H)),
            wfull((H, H)),
            wfull((H, 4 * H)),
            wfull((H, 4 * H)),
            wfull((IN, 4 * H)),
            wfull((H, 4 * H)),
            wfull((4 * H,)),
        ],
        out_specs=[bs3((1, SB, H)), bs3((1, SB, H))],
        out_shape=[out_sd, out_sd],
    )(x, h, c, g.reshape(B, 1, H), hbar, Wn_a, Wh, Wn, U, V_w, V_b)
    return new_h, new_c


# speedup vs baseline: 11.5334x; 1.1949x over previous
import functools

import jax
import jax.numpy as jnp
from jax import lax
from jax.experimental import pallas as pl
from jax.experimental.pallas import tpu as pltpu
from jax.experimental.pallas import tpu_sc as plsc

_L = 16


def _make_sc_hbar(B, S, N, H, SB):
    NBLK = S // SB
    HC = H // _L

    @functools.partial(
        pl.kernel,
        out_type=jax.ShapeDtypeStruct((B, S, H), jnp.float32),
        mesh=plsc.VectorSubcoreMesh(core_axis_name="c", subcore_axis_name="s"),
        compiler_params=pltpu.CompilerParams(needs_layout_passes=False),
        scratch_types=[
            pltpu.VMEM((S + 1, H), jnp.float32),
            pltpu.VMEM((SB, N), jnp.int32),
            pltpu.VMEM((SB, N), jnp.int32),
            pltpu.VMEM((SB, H), jnp.float32),
        ],
    )
    def sc_hbar(h3, idx3, mask3, out, table, idxv, maskv, outv):
        b = lax.axis_index("s") * 2 + lax.axis_index("c")
        zero16 = jnp.zeros((_L,), jnp.float32)
        for k in range(HC):
            table[0, pl.ds(k * _L, _L)] = zero16
        pltpu.sync_copy(h3.at[b], table.at[pl.ds(1, S)])
        iota = lax.iota(jnp.int32, _L)
        ramps = [iota + j * _L for j in range(HC)]

        def blk_body(blk, carry):
            pltpu.sync_copy(idx3.at[b, pl.ds(blk * SB, SB)], idxv)
            pltpu.sync_copy(mask3.at[b, pl.ds(blk * SB, SB)], maskv)

            def s_body(s, c):
                acc = [zero16] * HC
                for half in range(N // _L):
                    e = (idxv[s, pl.ds(half * _L, _L)]
                         * maskv[s, pl.ds(half * _L, _L)])
                    for n in range(_L):
                        row = jnp.take_along_axis(
                            e, jnp.full((_L,), n, jnp.int32), axis=0,
                            mode="promise_in_bounds")
                        for j in range(HC):
                            acc[j] = acc[j] + plsc.load_gather(
                                table, [row, ramps[j]])
                for j in range(HC):
                    outv[s, pl.ds(j * _L, _L)] = acc[j] * (1.0 / N)
                return c

            lax.fori_loop(0, SB, s_body, 0)
            pltpu.sync_copy(outv, out.at[b, pl.ds(blk * SB, SB)])
            return carry

        lax.fori_loop(0, NBLK, blk_body, 0)

    return sc_hbar


def _tc_body(x_ref, h_ref, c_ref, g_ref, hb_ref, Wn_a_ref, Wh_ref, Wn_ref,
             U_ref, V_w_ref, V_b_ref, nh_ref, nc_ref):
    H = h_ref.shape[-1]
    f32 = jnp.float32
    hn = jnp.dot(hb_ref[0], Wn_a_ref[...], preferred_element_type=f32)
    gates = (
        jnp.dot(h_ref[0], Wh_ref[...], preferred_element_type=f32)
        + jnp.dot(x_ref[0], U_ref[...], preferred_element_type=f32)
        + jnp.dot(hn, Wn_ref[...], preferred_element_type=f32)
        + jnp.dot(g_ref[0], V_w_ref[...], preferred_element_type=f32)
        + V_b_ref[...][None, :]
    )
    i_g = gates[:, 0:H]
    f_g = gates[:, H:2 * H]
    o_g = gates[:, 2 * H:3 * H]
    u_g = gates[:, 3 * H:4 * H]
    nc = jax.nn.sigmoid(f_g) * c_ref[0] + jax.nn.sigmoid(i_g) * jnp.tanh(u_g)
    nh_ref[0] = jax.nn.sigmoid(o_g) * jnp.tanh(nc)
    nc_ref[0] = nc


def kernel(x, h, c, g, Wh_a, Wn_a, U_a, u_w, u_b, Va_w, Va_b, Wh, Wn, U,
           V_w, V_b, neighbor_index, neighbor_mask):
    B, S, N = neighbor_index.shape
    H = h.shape[-1]
    IN = x.shape[-1]
    SB = 128
    NBLK = S // SB

    hbar = _make_sc_hbar(B, S, N, H, SB)(h, neighbor_index, neighbor_mask)

    grid = (B, NBLK)
    out_sd = jax.ShapeDtypeStruct((B, S, H), jnp.float32)
    bs3 = lambda shp: pl.BlockSpec(shp, lambda b, sb: (b, sb, 0))
    wfull = lambda shp: pl.BlockSpec(shp, lambda b, sb: tuple(0 for _ in shp))
    new_h, new_c = pl.pallas_call(
        _tc_body,
        grid=grid,
        in_specs=[
            bs3((1, SB, IN)),
            bs3((1, SB, H)),
            bs3((1, SB, H)),
            pl.BlockSpec((1, 1, H), lambda b, sb: (b, 0, 0)),
            bs3((1, SB, H)),
            wfull((H, H)),
            wfull((H, 4 * H)),
            wfull((H, 4 * H)),
            wfull((IN, 4 * H)),
            wfull((H, 4 * H)),
            wfull((4 * H,)),
        ],
        out_specs=[bs3((1, SB, H)), bs3((1, SB, H))],
        out_shape=[out_sd, out_sd],
    )(x, h, c, g.reshape(B, 1, H), hbar, Wn_a, Wh, Wn, U, V_w, V_b)
    return new_h, new_c


# speedup vs baseline: 13.3289x; 1.1557x over previous
import functools

import jax
import jax.numpy as jnp
from jax import lax
from jax.experimental import pallas as pl
from jax.experimental.pallas import tpu as pltpu
from jax.experimental.pallas import tpu_sc as plsc

_L = 16


def _make_sc_hbar(B, S, N, H, SB):
    NBLK = S // SB
    W = H // 2
    WC = W // _L
    f32 = jnp.float32

    @functools.partial(
        pl.kernel,
        out_type=(jax.ShapeDtypeStruct((B, S, H // 2), f32),
                  jax.ShapeDtypeStruct((B, S, H // 2), f32)),
        mesh=plsc.VectorSubcoreMesh(core_axis_name="c", subcore_axis_name="s"),
        compiler_params=pltpu.CompilerParams(needs_layout_passes=False),
        scratch_types=[
            pltpu.VMEM(((S + 2) * W,), jnp.int32),
            pltpu.VMEM((SB, N), jnp.int32),
            pltpu.VMEM((SB, N), jnp.int32),
            pltpu.VMEM((SB, H // 2), f32),
            pltpu.VMEM((SB, H // 2), f32),
        ],
    )
    def sc_hbar(h_pairs, idx3, mask3, out_e, out_o, tflat, idxv, maskv,
                outv_e, outv_o):
        b = lax.axis_index("s") * 2 + lax.axis_index("c")
        zero16i = jnp.zeros((_L,), jnp.int32)
        zero16 = jnp.zeros((_L,), f32)
        himask = jnp.full((_L,), -65536, jnp.int32)
        for k in range(WC):
            tflat[pl.ds(W + k * _L, _L)] = zero16i
        pltpu.sync_copy(h_pairs.at[b], tflat.at[pl.ds(2 * W, S * W)])

        def blk_body(blk, carry):
            pltpu.sync_copy(idx3.at[b, pl.ds(blk * SB, SB)], idxv)
            pltpu.sync_copy(mask3.at[b, pl.ds(blk * SB, SB)], maskv)

            def s_body(s, c):
                acc_e = [zero16] * WC
                acc_o = [zero16] * WC
                for half in range(N // _L):
                    e = (idxv[s, pl.ds(half * _L, _L)]
                         * maskv[s, pl.ds(half * _L, _L)])
                    base = e * W
                    for n in range(_L):
                        row = base[n]
                        for j in range(WC):
                            w = tflat[pl.ds(row + (W + j * _L), _L)]
                            lo = plsc.bitcast(w << 16, f32)
                            hi = plsc.bitcast(w & himask, f32)
                            acc_e[j] = acc_e[j] + lo
                            acc_o[j] = acc_o[j] + hi
                for j in range(WC):
                    outv_e[s, pl.ds(j * _L, _L)] = acc_e[j] * (1.0 / N)
                    outv_o[s, pl.ds(j * _L, _L)] = acc_o[j] * (1.0 / N)
                return c

            lax.fori_loop(0, SB, s_body, 0)
            pltpu.sync_copy(outv_e, out_e.at[b, pl.ds(blk * SB, SB)])
            pltpu.sync_copy(outv_o, out_o.at[b, pl.ds(blk * SB, SB)])
            return carry

        lax.fori_loop(0, NBLK, blk_body, 0)

    return sc_hbar


def _tc_body(x_ref, h_ref, c_ref, g_ref, hbe_ref, hbo_ref, Wn_a_e_ref,
             Wn_a_o_ref, Wh_ref, Wn_ref, U_ref, V_w_ref, V_b_ref,
             nh_ref, nc_ref):
    H = h_ref.shape[-1]
    f32 = jnp.float32
    hn = (jnp.dot(hbe_ref[0], Wn_a_e_ref[...], preferred_element_type=f32)
          + jnp.dot(hbo_ref[0], Wn_a_o_ref[...], preferred_element_type=f32))
    gates = (
        jnp.dot(h_ref[0], Wh_ref[...], preferred_element_type=f32)
        + jnp.dot(x_ref[0], U_ref[...], preferred_element_type=f32)
        + jnp.dot(hn, Wn_ref[...], preferred_element_type=f32)
        + jnp.dot(g_ref[0], V_w_ref[...], preferred_element_type=f32)
        + V_b_ref[...][None, :]
    )
    i_g = gates[:, 0:H]
    f_g = gates[:, H:2 * H]
    o_g = gates[:, 2 * H:3 * H]
    u_g = gates[:, 3 * H:4 * H]
    nc = jax.nn.sigmoid(f_g) * c_ref[0] + jax.nn.sigmoid(i_g) * jnp.tanh(u_g)
    nh_ref[0] = jax.nn.sigmoid(o_g) * jnp.tanh(nc)
    nc_ref[0] = nc


def kernel(x, h, c, g, Wh_a, Wn_a, U_a, u_w, u_b, Va_w, Va_b, Wh, Wn, U,
           V_w, V_b, neighbor_index, neighbor_mask):
    B, S, N = neighbor_index.shape
    H = h.shape[-1]
    IN = x.shape[-1]
    SB = 128
    NBLK = S // SB

    h_pairs = jax.lax.bitcast_convert_type(
        h.astype(jnp.bfloat16).reshape(B, S * H // 2, 2), jnp.int32)
    hbar_e, hbar_o = _make_sc_hbar(B, S, N, H, SB)(
        h_pairs, neighbor_index, neighbor_mask)

    grid = (B, NBLK)
    out_sd = jax.ShapeDtypeStruct((B, S, H), jnp.float32)
    bs3 = lambda shp: pl.BlockSpec(shp, lambda b, sb: (b, sb, 0))
    wfull = lambda shp: pl.BlockSpec(shp, lambda b, sb: tuple(0 for _ in shp))
    new_h, new_c = pl.pallas_call(
        _tc_body,
        grid=grid,
        in_specs=[
            bs3((1, SB, IN)),
            bs3((1, SB, H)),
            bs3((1, SB, H)),
            pl.BlockSpec((1, 1, H), lambda b, sb: (b, 0, 0)),
            bs3((1, SB, H // 2)),
            bs3((1, SB, H // 2)),
            wfull((H // 2, H)),
            wfull((H // 2, H)),
            wfull((H, 4 * H)),
            wfull((H, 4 * H)),
            wfull((IN, 4 * H)),
            wfull((H, 4 * H)),
            wfull((4 * H,)),
        ],
        out_specs=[bs3((1, SB, H)), bs3((1, SB, H))],
        out_shape=[out_sd, out_sd],
    )(x, h, c, g.reshape(B, 1, H), hbar_e, hbar_o,
      Wn_a[0::2], Wn_a[1::2], Wh, Wn, U, V_w, V_b)
    return new_h, new_c


# speedup vs baseline: 35.1764x; 2.6391x over previous
import functools

import jax
import jax.numpy as jnp
from jax import lax
from jax.experimental import pallas as pl
from jax.experimental.pallas import tpu as pltpu
from jax.experimental.pallas import tpu_sc as plsc

_L = 16


def _make_sc_hbar(B, S, N, H, SB):
    NBLK = S // SB
    W = H // 2
    WC = W // _L
    f32 = jnp.float32

    @functools.partial(
        pl.kernel,
        out_type=jax.ShapeDtypeStruct((B, S, H), f32),
        mesh=plsc.VectorSubcoreMesh(core_axis_name="c", subcore_axis_name="s"),
        compiler_params=pltpu.CompilerParams(needs_layout_passes=False),
        scratch_types=[
            pltpu.VMEM(((S + 1) * W,), jnp.int32),
            pltpu.VMEM((SB, N), jnp.int32),
            pltpu.VMEM((SB, N), jnp.int32),
            pltpu.VMEM((SB, H), f32),
            pltpu.VMEM((SB * H,), f32),
        ],
    )
    def sc_hbar(h2, idx3, mask3, out, tflat, idxv, maskv, outv, hstage):
        b = lax.axis_index("s") * 2 + lax.axis_index("c")
        zero16i = jnp.zeros((_L,), jnp.int32)
        zero16 = jnp.zeros((_L,), f32)
        for k in range(WC):
            tflat[pl.ds(k * _L, _L)] = zero16i

        def pack_blk(blk, carry):
            pltpu.sync_copy(h2.at[b, pl.ds(blk * SB * H, SB * H)], hstage)

            def pack_u(u, c):
                c0 = hstage[pl.ds(u * 2 * _L, _L)]
                c1 = hstage[pl.ds(u * 2 * _L + _L, _L)]
                w = plsc.bitcast(
                    plsc.pack(c0, c1, format=plsc.PackFormat.INTERLEAVED),
                    jnp.int32)
                tflat[pl.ds(W + blk * SB * W + u * _L, _L)] = w
                return c

            lax.fori_loop(0, SB * H // (2 * _L), pack_u, 0)
            return carry

        lax.fori_loop(0, NBLK, pack_blk, 0)

        def blk_body(blk, carry):
            pltpu.sync_copy(idx3.at[b, pl.ds(blk * SB, SB)], idxv)
            pltpu.sync_copy(mask3.at[b, pl.ds(blk * SB, SB)], maskv)

            def s_body(s, c):
                acc_lo = [zero16] * WC
                acc_hi = [zero16] * WC
                for half in range(N // _L):
                    e = (idxv[s, pl.ds(half * _L, _L)]
                         * maskv[s, pl.ds(half * _L, _L)])
                    base = e * W
                    for n in range(_L):
                        row = base[n]
                        for j in range(WC):
                            w = tflat[pl.ds(row + j * _L, _L)]
                            lo = plsc.bitcast(w << 16, f32)
                            hi = plsc.bitcast(w, f32)
                            acc_lo[j] = acc_lo[j] + lo
                            acc_hi[j] = acc_hi[j] + hi
                for j in range(WC):
                    outv[s, pl.ds(2 * j * _L, _L)] = acc_lo[j] * (1.0 / N)
                    outv[s, pl.ds((2 * j + 1) * _L, _L)] = acc_hi[j] * (1.0 / N)
                return c

            lax.fori_loop(0, SB, s_body, 0)
            pltpu.sync_copy(outv, out.at[b, pl.ds(blk * SB, SB)])
            return carry

        lax.fori_loop(0, NBLK, blk_body, 0)

    return sc_hbar


def _tc_body(x_ref, h_ref, c_ref, g_ref, hb_ref, Wn_a_ref, Wh_ref, Wn_ref,
             U_ref, V_w_ref, V_b_ref, nh_ref, nc_ref):
    H = h_ref.shape[-1]
    f32 = jnp.float32
    bf16 = jnp.bfloat16
    hn = jnp.dot(hb_ref[0].astype(bf16), Wn_a_ref[...].astype(bf16),
                 preferred_element_type=f32)
    gates = (
        jnp.dot(h_ref[0].astype(bf16), Wh_ref[...].astype(bf16),
                preferred_element_type=f32)
        + jnp.dot(x_ref[0].astype(bf16), U_ref[...].astype(bf16),
                  preferred_element_type=f32)
        + jnp.dot(hn.astype(bf16), Wn_ref[...].astype(bf16),
                  preferred_element_type=f32)
        + jnp.dot(g_ref[0].astype(bf16), V_w_ref[...].astype(bf16),
                  preferred_element_type=f32)
        + V_b_ref[...][None, :]
    )
    i_g = gates[:, 0:H]
    f_g = gates[:, H:2 * H]
    o_g = gates[:, 2 * H:3 * H]
    u_g = gates[:, 3 * H:4 * H]
    nc = jax.nn.sigmoid(f_g) * c_ref[0] + jax.nn.sigmoid(i_g) * jnp.tanh(u_g)
    nh_ref[0] = jax.nn.sigmoid(o_g) * jnp.tanh(nc)
    nc_ref[0] = nc


def kernel(x, h, c, g, Wh_a, Wn_a, U_a, u_w, u_b, Va_w, Va_b, Wh, Wn, U,
           V_w, V_b, neighbor_index, neighbor_mask):
    B, S, N = neighbor_index.shape
    H = h.shape[-1]
    IN = x.shape[-1]
    SB = 128
    NBLK = S // SB

    hbar = _make_sc_hbar(B, S, N, H, SB)(
        h.reshape(B, S * H), neighbor_index, neighbor_mask)

    STC = 512
    grid = (B, S // STC)
    out_sd = jax.ShapeDtypeStruct((B, S, H), jnp.float32)
    bs3 = lambda shp: pl.BlockSpec(shp, lambda b, sb: (b, sb, 0))
    wfull = lambda shp: pl.BlockSpec(shp, lambda b, sb: tuple(0 for _ in shp))
    new_h, new_c = pl.pallas_call(
        _tc_body,
        grid=grid,
        in_specs=[
            bs3((1, STC, IN)),
            bs3((1, STC, H)),
            bs3((1, STC, H)),
            pl.BlockSpec((1, 1, H), lambda b, sb: (b, 0, 0)),
            bs3((1, STC, H)),
            wfull((H, H)),
            wfull((H, 4 * H)),
            wfull((H, 4 * H)),
            wfull((IN, 4 * H)),
            wfull((H, 4 * H)),
            wfull((4 * H,)),
        ],
        out_specs=[bs3((1, STC, H)), bs3((1, STC, H))],
        out_shape=[out_sd, out_sd],
        compiler_params=pltpu.CompilerParams(
            dimension_semantics=("parallel", "parallel")),
    )(x, h, c, g.reshape(B, 1, H), hbar, Wn_a, Wh, Wn, U, V_w, V_b)
    return new_h, new_c
